# baseline (speedup 1.0000x reference)
import functools

import jax
import jax.numpy as jnp
from jax import lax
from jax.experimental import pallas as pl
from jax.experimental.pallas import tpu as pltpu
from jax.experimental.pallas import tpu_sc as plsc

N_NODES_C = 10000
HIDDEN_C = 64
OUT_C = 128
EDGE_BLOCK = 1280

NUM_WORKERS = 32
CHUNK = 128
CHUNKS_PER_W = 40
EDGES_PER_W = CHUNK * CHUNKS_PER_W
E_PAD = NUM_WORKERS * EDGES_PER_W

_INTERPRET = False


def _sc_gather(src_idx3, node_features):
    mesh = plsc.VectorSubcoreMesh(core_axis_name="c", subcore_axis_name="s")

    @functools.partial(
        pl.kernel, mesh=mesh,
        out_type=jax.ShapeDtypeStruct((E_PAD, HIDDEN_C), jnp.float32),
        scratch_types=[
            pltpu.VMEM((CHUNKS_PER_W, CHUNK), jnp.int32),
            pltpu.VMEM((CHUNK, HIDDEN_C), jnp.float32),
            pltpu.VMEM((CHUNK, HIDDEN_C), jnp.float32),
            pltpu.SemaphoreType.DMA,
            pltpu.SemaphoreType.DMA,
        ],
        compiler_params=pltpu.CompilerParams(use_tc_tiling_on_sc=False),
    )
    def k(idx_hbm, nf_hbm, out_hbm, idx_v, rows0, rows1, sem0, sem1):
        wid = lax.axis_index("s") * 2 + lax.axis_index("c")
        pltpu.sync_copy(idx_hbm.at[wid], idx_v)
        base = wid * EDGES_PER_W
        bufs = (rows0, rows1)
        sems = (sem0, sem1)
        pltpu.async_copy(nf_hbm.at[idx_v.at[0]], rows0, sem0)
        pltpu.async_copy(nf_hbm.at[idx_v.at[1]], rows1, sem1)

        def pair(t, carry):
            for bslot in range(2):
                j = t * 2 + bslot
                buf, sem = bufs[bslot], sems[bslot]
                pltpu.make_async_copy(nf_hbm.at[idx_v.at[0]], buf, sem).wait()
                pltpu.sync_copy(buf, out_hbm.at[pl.ds(base + j * CHUNK, CHUNK)])

                @pl.when(j + 2 < CHUNKS_PER_W)
                def _():
                    pltpu.async_copy(nf_hbm.at[idx_v.at[j + 2]], buf, sem)
            return carry

        lax.fori_loop(0, CHUNKS_PER_W // 2, pair, 0)

    return k(src_idx3, node_features)


NACC = 10240


def _sc_scatter_add(tgt_idx3, zz, zeros):
    mesh = plsc.VectorSubcoreMesh(core_axis_name="c", subcore_axis_name="s")
    stripe = NACC // 16

    @functools.partial(
        pl.kernel, mesh=mesh,
        out_type=jax.ShapeDtypeStruct((2, NACC, OUT_C), jnp.float32),
        scratch_types=[
            pltpu.VMEM((CHUNKS_PER_W, CHUNK), jnp.int32),
            pltpu.VMEM((CHUNK, OUT_C), jnp.float32),
            pltpu.VMEM((CHUNK, OUT_C), jnp.float32),
            pltpu.VMEM_SHARED((NACC, OUT_C), jnp.float32),
            pltpu.SemaphoreType.DMA,
            pltpu.SemaphoreType.DMA,
        ],
        compiler_params=pltpu.CompilerParams(use_tc_tiling_on_sc=True),
    )
    def k(idx_hbm, z_hbm, zeros_hbm, out_hbm, idx_v, z0, z1, acc, sem0, sem1):
        c = lax.axis_index("c")
        s = lax.axis_index("s")
        wid = s * 2 + c
        pltpu.sync_copy(zeros_hbm.at[pl.ds(s * stripe, stripe)],
                        acc.at[pl.ds(s * stripe, stripe)])
        pltpu.sync_copy(idx_hbm.at[wid], idx_v)
        plsc.subcore_barrier()
        base = wid * EDGES_PER_W
        bufs = (z0, z1)
        sems = (sem0, sem1)
        pltpu.async_copy(z_hbm.at[pl.ds(base, CHUNK)], z0, sem0)
        pltpu.async_copy(z_hbm.at[pl.ds(base + CHUNK, CHUNK)], z1, sem1)

        def pair(t, carry):
            for bslot in range(2):
                j = t * 2 + bslot
                buf, sem = bufs[bslot], sems[bslot]
                pltpu.make_async_copy(z_hbm.at[pl.ds(base, CHUNK)], buf,
                                      sem).wait()
                pltpu.sync_copy(buf, acc.at[idx_v.at[j]], add=True)

                @pl.when(j + 2 < CHUNKS_PER_W)
                def _():
                    pltpu.async_copy(
                        z_hbm.at[pl.ds(base + (j + 2) * CHUNK, CHUNK)],
                        buf, sem)
            return carry

        lax.fori_loop(0, CHUNKS_PER_W // 2, pair, 0)
        plsc.subcore_barrier()
        pltpu.sync_copy(acc.at[pl.ds(s * stripe, stripe)],
                        out_hbm.at[c, pl.ds(s * stripe, stripe)])

    return k(tgt_idx3, zz, zeros)


def _combine_body(p0_ref, p1_ref, b_ref, o_ref):
    o_ref[...] = p0_ref[0] + p1_ref[0] + b_ref[...]


def _combine(partials, b):
    nb = 5
    blk = N_NODES_C // nb
    return pl.pallas_call(
        _combine_body,
        grid=(nb,),
        in_specs=[
            pl.BlockSpec((1, blk, OUT_C), lambda i: (0, i, 0)),
            pl.BlockSpec((1, blk, OUT_C), lambda i: (1, i, 0)),
            pl.BlockSpec((1, OUT_C), lambda i: (0, 0)),
        ],
        out_specs=pl.BlockSpec((blk, OUT_C), lambda i: (i, 0)),
        out_shape=jax.ShapeDtypeStruct((N_NODES_C, OUT_C), jnp.float32),
        interpret=_INTERPRET,
    )(partials, partials, b.reshape(1, OUT_C))


def _edge_matmul_body(g_ref, evt_ref, w2t_ref, tconst_ref, econst_ref, z_ref):
    x = evt_ref[0:1, :]
    y = evt_ref[1:2, :]
    z = evt_ref[2:3, :]
    n2 = x * x + y * y + z * z
    inv = jax.lax.rsqrt(jnp.maximum(n2, 1e-24))
    x = x * inv
    y = y * inv
    z = z * inv
    x2, y2, z2 = x * x, y * y, z * z
    sh = [
        0.28209479177387814 * jnp.ones_like(x),
        0.4886025119029199 * y,
        0.4886025119029199 * z,
        0.4886025119029199 * x,
        1.0925484305920792 * x * y,
        1.0925484305920792 * y * z,
        0.31539156525252005 * (3.0 * z2 - 1.0),
        1.0925484305920792 * x * z,
        0.5462742152960396 * (x2 - y2),
        0.5900435899266435 * y * (3.0 * x2 - y2),
        2.890611442640554 * x * y * z,
        0.4570457994644658 * y * (5.0 * z2 - 1.0),
        0.3731763325901154 * z * (5.0 * z2 - 3.0),
        0.4570457994644658 * x * (5.0 * z2 - 1.0),
        1.445305721320277 * z * (x2 - y2),
        0.5900435899266435 * x * (x2 - 3.0 * y2),
    ]
    sht = jnp.concatenate(sh, axis=0)
    shc = sht.T.astype(jnp.bfloat16)
    dims = (((1,), (0,)), ((), ()))
    half = EDGE_BLOCK // 4
    for h in range(4):
        rows = pl.ds(h * half, half)
        g = g_ref[rows, :].astype(jnp.bfloat16)
        gexp = jax.lax.dot_general(g, tconst_ref[...], dimension_numbers=dims,
                                   preferred_element_type=jnp.float32)
        shexp = jax.lax.dot_general(shc[h * half:(h + 1) * half, :],
                                    econst_ref[...], dimension_numbers=dims,
                                    preferred_element_type=jnp.float32)
        msgs = gexp.astype(jnp.bfloat16) * shexp.astype(jnp.bfloat16)
        z_ref[rows, :] = jax.lax.dot_general(
            msgs, w2t_ref[...],
            dimension_numbers=dims,
            preferred_element_type=jnp.float32,
        )


def _edge_matmul(gathered, evt, w2t, tconst, econst):
    e_pad = gathered.shape[0]
    grid = e_pad // EDGE_BLOCK
    return pl.pallas_call(
        _edge_matmul_body,
        grid=(grid,),
        in_specs=[
            pl.BlockSpec((EDGE_BLOCK, HIDDEN_C), lambda i: (i, 0)),
            pl.BlockSpec((3, EDGE_BLOCK), lambda i: (0, i)),
            pl.BlockSpec((HIDDEN_C * 16, OUT_C), lambda i: (0, 0)),
            pl.BlockSpec((HIDDEN_C, HIDDEN_C * 16), lambda i: (0, 0)),
            pl.BlockSpec((16, HIDDEN_C * 16), lambda i: (0, 0)),
        ],
        out_specs=pl.BlockSpec((EDGE_BLOCK, OUT_C), lambda i: (i, 0)),
        out_shape=jax.ShapeDtypeStruct((e_pad, OUT_C), jnp.float32),
        interpret=_INTERPRET,
    )(gathered, evt, w2t, tconst, econst)


def kernel(node_features, edge_vectors, edge_index, W, b):
    n_atoms = node_features.shape[0]
    n_edges = edge_index.shape[1]
    sources = edge_index[0]
    targets = edge_index[1]
    w2t = W.reshape(OUT_C, HIDDEN_C, 16).transpose(2, 1, 0).reshape(
        16 * HIDDEN_C, OUT_C).astype(jnp.bfloat16)
    src_pad = jnp.pad(sources, (0, E_PAD - n_edges)).reshape(
        NUM_WORKERS, CHUNKS_PER_W, CHUNK)
    tgt_pad = jnp.pad(targets, (0, E_PAD - n_edges),
                      constant_values=n_atoms).reshape(
        NUM_WORKERS, CHUNKS_PER_W, CHUNK)
    evt = jnp.pad(edge_vectors, ((0, E_PAD - n_edges), (0, 0))).T
    lm = jnp.arange(16 * HIDDEN_C) // HIDDEN_C
    ch = jnp.arange(16 * HIDDEN_C) % HIDDEN_C
    tconst = (jnp.arange(HIDDEN_C)[:, None] == ch[None, :]).astype(jnp.bfloat16)
    econst = (jnp.arange(16)[:, None] == lm[None, :]).astype(jnp.bfloat16)
    gathered = _sc_gather(src_pad, node_features)
    zz = _edge_matmul(gathered, evt, w2t, tconst, econst)
    zeros = jnp.zeros((NACC, OUT_C), jnp.float32)
    partials = _sc_scatter_add(tgt_pad, zz, zeros)
    return _combine(partials, b)


# speedup vs baseline: 1.0373x; 1.0373x over previous
import functools

import jax
import jax.numpy as jnp
from jax import lax
from jax.experimental import pallas as pl
from jax.experimental.pallas import tpu as pltpu
from jax.experimental.pallas import tpu_sc as plsc

N_NODES_C = 10000
HIDDEN_C = 64
OUT_C = 128
EDGE_BLOCK = 2560

NUM_WORKERS = 32
CHUNK = 128
CHUNKS_PER_W = 40
EDGES_PER_W = CHUNK * CHUNKS_PER_W
E_PAD = NUM_WORKERS * EDGES_PER_W

_INTERPRET = False


def _sc_gather(src_idx3, node_features, nchunks):
    mesh = plsc.VectorSubcoreMesh(core_axis_name="c", subcore_axis_name="s")
    per_w = nchunks * CHUNK

    @functools.partial(
        pl.kernel, mesh=mesh,
        out_type=jax.ShapeDtypeStruct((NUM_WORKERS * per_w, HIDDEN_C),
                                      jnp.float32),
        scratch_types=[
            pltpu.VMEM((nchunks, CHUNK), jnp.int32),
            pltpu.VMEM((CHUNK, HIDDEN_C), jnp.float32),
            pltpu.VMEM((CHUNK, HIDDEN_C), jnp.float32),
            pltpu.SemaphoreType.DMA,
            pltpu.SemaphoreType.DMA,
        ],
        compiler_params=pltpu.CompilerParams(use_tc_tiling_on_sc=False),
    )
    def k(idx_hbm, nf_hbm, out_hbm, idx_v, rows0, rows1, sem0, sem1):
        wid = lax.axis_index("s") * 2 + lax.axis_index("c")
        pltpu.sync_copy(idx_hbm.at[wid], idx_v)
        base = wid * per_w
        bufs = (rows0, rows1)
        sems = (sem0, sem1)
        pltpu.async_copy(nf_hbm.at[idx_v.at[0]], rows0, sem0)
        pltpu.async_copy(nf_hbm.at[idx_v.at[1]], rows1, sem1)

        def pair(t, carry):
            for bslot in range(2):
                j = t * 2 + bslot
                buf, sem = bufs[bslot], sems[bslot]
                pltpu.make_async_copy(nf_hbm.at[idx_v.at[0]], buf, sem).wait()
                pltpu.sync_copy(buf, out_hbm.at[pl.ds(base + j * CHUNK, CHUNK)])

                @pl.when(j + 2 < nchunks)
                def _():
                    pltpu.async_copy(nf_hbm.at[idx_v.at[j + 2]], buf, sem)
            return carry

        lax.fori_loop(0, nchunks // 2, pair, 0)

    return k(src_idx3, node_features)


NACC = 10240


def _sc_scatter_add(tgt_idx3, zz, zeros, nchunks):
    mesh = plsc.VectorSubcoreMesh(core_axis_name="c", subcore_axis_name="s")
    stripe = NACC // 16
    per_w = nchunks * CHUNK

    @functools.partial(
        pl.kernel, mesh=mesh,
        out_type=jax.ShapeDtypeStruct((2, NACC, OUT_C), jnp.float32),
        scratch_types=[
            pltpu.VMEM((nchunks, CHUNK), jnp.int32),
            pltpu.VMEM((CHUNK, OUT_C), jnp.float32),
            pltpu.VMEM((CHUNK, OUT_C), jnp.float32),
            pltpu.VMEM_SHARED((NACC, OUT_C), jnp.float32),
            pltpu.SemaphoreType.DMA,
            pltpu.SemaphoreType.DMA,
        ],
        compiler_params=pltpu.CompilerParams(use_tc_tiling_on_sc=True),
    )
    def k(idx_hbm, z_hbm, zeros_hbm, out_hbm, idx_v, z0, z1, acc, sem0, sem1):
        c = lax.axis_index("c")
        s = lax.axis_index("s")
        wid = s * 2 + c
        pltpu.sync_copy(zeros_hbm.at[pl.ds(s * stripe, stripe)],
                        acc.at[pl.ds(s * stripe, stripe)])
        pltpu.sync_copy(idx_hbm.at[wid], idx_v)
        plsc.subcore_barrier()
        base = wid * per_w
        bufs = (z0, z1)
        sems = (sem0, sem1)
        pltpu.async_copy(z_hbm.at[pl.ds(base, CHUNK)], z0, sem0)
        pltpu.async_copy(z_hbm.at[pl.ds(base + CHUNK, CHUNK)], z1, sem1)

        def pair(t, carry):
            for bslot in range(2):
                j = t * 2 + bslot
                buf, sem = bufs[bslot], sems[bslot]
                pltpu.make_async_copy(z_hbm.at[pl.ds(base, CHUNK)], buf,
                                      sem).wait()
                pltpu.sync_copy(buf, acc.at[idx_v.at[j]], add=True)

                @pl.when(j + 2 < nchunks)
                def _():
                    pltpu.async_copy(
                        z_hbm.at[pl.ds(base + (j + 2) * CHUNK, CHUNK)],
                        buf, sem)
            return carry

        lax.fori_loop(0, nchunks // 2, pair, 0)
        plsc.subcore_barrier()
        pltpu.sync_copy(acc.at[pl.ds(s * stripe, stripe)],
                        out_hbm.at[c, pl.ds(s * stripe, stripe)])

    return k(tgt_idx3, zz, zeros)


PHASE_CHUNKS = CHUNKS_PER_W // 2
PHASE_EDGES = NUM_WORKERS * PHASE_CHUNKS * CHUNK


def _combine_body(pa0_ref, pa1_ref, pb0_ref, pb1_ref, b_ref, o_ref):
    o_ref[...] = ((pa0_ref[0] + pa1_ref[0]) + (pb0_ref[0] + pb1_ref[0])
                  + b_ref[...])


def _combine(pa, pb, b):
    nb = 5
    blk = N_NODES_C // nb
    return pl.pallas_call(
        _combine_body,
        grid=(nb,),
        in_specs=[
            pl.BlockSpec((1, blk, OUT_C), lambda i: (0, i, 0)),
            pl.BlockSpec((1, blk, OUT_C), lambda i: (1, i, 0)),
            pl.BlockSpec((1, blk, OUT_C), lambda i: (0, i, 0)),
            pl.BlockSpec((1, blk, OUT_C), lambda i: (1, i, 0)),
            pl.BlockSpec((1, OUT_C), lambda i: (0, 0)),
        ],
        out_specs=pl.BlockSpec((blk, OUT_C), lambda i: (i, 0)),
        out_shape=jax.ShapeDtypeStruct((N_NODES_C, OUT_C), jnp.float32),
        interpret=_INTERPRET,
    )(pa, pa, pb, pb, b.reshape(1, OUT_C))


def _edge_matmul_body(g_ref, evt_ref, w2t_ref, tconst_ref, econst_ref, z_ref):
    x = evt_ref[0:1, :]
    y = evt_ref[1:2, :]
    z = evt_ref[2:3, :]
    n2 = x * x + y * y + z * z
    inv = jax.lax.rsqrt(jnp.maximum(n2, 1e-24))
    x = x * inv
    y = y * inv
    z = z * inv
    x2, y2, z2 = x * x, y * y, z * z
    sh = [
        0.28209479177387814 * jnp.ones_like(x),
        0.4886025119029199 * y,
        0.4886025119029199 * z,
        0.4886025119029199 * x,
        1.0925484305920792 * x * y,
        1.0925484305920792 * y * z,
        0.31539156525252005 * (3.0 * z2 - 1.0),
        1.0925484305920792 * x * z,
        0.5462742152960396 * (x2 - y2),
        0.5900435899266435 * y * (3.0 * x2 - y2),
        2.890611442640554 * x * y * z,
        0.4570457994644658 * y * (5.0 * z2 - 1.0),
        0.3731763325901154 * z * (5.0 * z2 - 3.0),
        0.4570457994644658 * x * (5.0 * z2 - 1.0),
        1.445305721320277 * z * (x2 - y2),
        0.5900435899266435 * x * (x2 - 3.0 * y2),
    ]
    sht = jnp.concatenate(sh, axis=0)
    shc = sht.T.astype(jnp.bfloat16)
    dims = (((1,), (0,)), ((), ()))
    half = EDGE_BLOCK // 4
    for h in range(4):
        rows = pl.ds(h * half, half)
        g = g_ref[rows, :].astype(jnp.bfloat16)
        gexp = jax.lax.dot_general(g, tconst_ref[...], dimension_numbers=dims,
                                   preferred_element_type=jnp.float32)
        shexp = jax.lax.dot_general(shc[h * half:(h + 1) * half, :],
                                    econst_ref[...], dimension_numbers=dims,
                                    preferred_element_type=jnp.float32)
        msgs = gexp.astype(jnp.bfloat16) * shexp.astype(jnp.bfloat16)
        z_ref[rows, :] = jax.lax.dot_general(
            msgs, w2t_ref[...],
            dimension_numbers=dims,
            preferred_element_type=jnp.float32,
        )


def _edge_matmul(gathered, evt, w2t, tconst, econst, phase):
    n_rows = gathered.shape[0]
    grid = n_rows // EDGE_BLOCK
    return pl.pallas_call(
        _edge_matmul_body,
        grid=(grid,),
        in_specs=[
            pl.BlockSpec((EDGE_BLOCK, HIDDEN_C), lambda i: (i, 0)),
            pl.BlockSpec((3, EDGE_BLOCK), lambda i: (0, 2 * i + phase)),
            pl.BlockSpec((HIDDEN_C * 16, OUT_C), lambda i: (0, 0)),
            pl.BlockSpec((HIDDEN_C, HIDDEN_C * 16), lambda i: (0, 0)),
            pl.BlockSpec((16, HIDDEN_C * 16), lambda i: (0, 0)),
        ],
        out_specs=pl.BlockSpec((EDGE_BLOCK, OUT_C), lambda i: (i, 0)),
        out_shape=jax.ShapeDtypeStruct((n_rows, OUT_C), jnp.float32),
        interpret=_INTERPRET,
    )(gathered, evt, w2t, tconst, econst)


def kernel(node_features, edge_vectors, edge_index, W, b):
    n_atoms = node_features.shape[0]
    n_edges = edge_index.shape[1]
    sources = edge_index[0]
    targets = edge_index[1]
    w2t = W.reshape(OUT_C, HIDDEN_C, 16).transpose(2, 1, 0).reshape(
        16 * HIDDEN_C, OUT_C).astype(jnp.bfloat16)
    src_pad = jnp.pad(sources, (0, E_PAD - n_edges)).reshape(
        NUM_WORKERS, CHUNKS_PER_W, CHUNK)
    tgt_pad = jnp.pad(targets, (0, E_PAD - n_edges),
                      constant_values=n_atoms).reshape(
        NUM_WORKERS, CHUNKS_PER_W, CHUNK)
    evt = jnp.pad(edge_vectors, ((0, E_PAD - n_edges), (0, 0))).T
    lm = jnp.arange(16 * HIDDEN_C) // HIDDEN_C
    ch = jnp.arange(16 * HIDDEN_C) % HIDDEN_C
    tconst = (jnp.arange(HIDDEN_C)[:, None] == ch[None, :]).astype(jnp.bfloat16)
    econst = (jnp.arange(16)[:, None] == lm[None, :]).astype(jnp.bfloat16)
    zeros = jnp.zeros((NACC, OUT_C), jnp.float32)
    ga = _sc_gather(src_pad[:, :PHASE_CHUNKS], node_features, PHASE_CHUNKS)
    za = _edge_matmul(ga, evt, w2t, tconst, econst, 0)
    gb = _sc_gather(src_pad[:, PHASE_CHUNKS:], node_features, PHASE_CHUNKS)
    pa = _sc_scatter_add(tgt_pad[:, :PHASE_CHUNKS], za, zeros, PHASE_CHUNKS)
    zb = _edge_matmul(gb, evt, w2t, tconst, econst, 1)
    pb = _sc_scatter_add(tgt_pad[:, PHASE_CHUNKS:], zb, zeros, PHASE_CHUNKS)
    return _combine(pa, pb, b)


# speedup vs baseline: 1.0774x; 1.0387x over previous
import functools

import jax
import jax.numpy as jnp
from jax import lax
from jax.experimental import pallas as pl
from jax.experimental.pallas import tpu as pltpu
from jax.experimental.pallas import tpu_sc as plsc

N_NODES_C = 10000
HIDDEN_C = 64
OUT_C = 128
EDGE_BLOCK = 2560

NUM_WORKERS = 32
CHUNK = 128
CHUNKS_PER_W = 40
EDGES_PER_W = CHUNK * CHUNKS_PER_W
E_PAD = NUM_WORKERS * EDGES_PER_W

_INTERPRET = False


def _sc_gather(src_idx3, node_features, nchunks):
    mesh = plsc.VectorSubcoreMesh(core_axis_name="c", subcore_axis_name="s")
    per_w = nchunks * CHUNK

    steal = 8

    @functools.partial(
        pl.kernel, mesh=mesh,
        out_type=jax.ShapeDtypeStruct((NUM_WORKERS * per_w, HIDDEN_C),
                                      jnp.float32),
        scratch_types=[
            pltpu.VMEM((nchunks + steal, CHUNK), jnp.int32),
            pltpu.VMEM((CHUNK, HIDDEN_C), jnp.float32),
            pltpu.VMEM((CHUNK, HIDDEN_C), jnp.float32),
            pltpu.SemaphoreType.DMA,
            pltpu.SemaphoreType.DMA,
        ],
        compiler_params=pltpu.CompilerParams(use_tc_tiling_on_sc=False),
    )
    def k(idx_hbm, nf_hbm, out_hbm, idx_v, rows0, rows1, sem0, sem1):
        c = lax.axis_index("c")
        s = lax.axis_index("s")
        wid = s * 2 + c
        pltpu.sync_copy(idx_hbm.at[wid], idx_v.at[pl.ds(0, nchunks)])

        @pl.when(c == 0)
        def _():
            pltpu.sync_copy(idx_hbm.at[wid + 1, pl.ds(nchunks - steal, steal)],
                            idx_v.at[pl.ds(nchunks, steal)])

        base = wid * per_w
        bufs = (rows0, rows1)
        sems = (sem0, sem1)
        pltpu.async_copy(nf_hbm.at[idx_v.at[0]], rows0, sem0)
        pltpu.async_copy(nf_hbm.at[idx_v.at[1]], rows1, sem1)

        def make_pair(total):
            def pair(t, carry):
                for bslot in range(2):
                    j = t * 2 + bslot
                    buf, sem = bufs[bslot], sems[bslot]
                    off = jnp.where(j < nchunks, base + j * CHUNK,
                                    (wid + 1) * per_w + (j - steal) * CHUNK)
                    pltpu.make_async_copy(nf_hbm.at[idx_v.at[0]], buf,
                                          sem).wait()
                    pltpu.sync_copy(buf, out_hbm.at[pl.ds(off, CHUNK)])

                    @pl.when(j + 2 < total)
                    def _():
                        pltpu.async_copy(nf_hbm.at[idx_v.at[j + 2]], buf, sem)
                return carry
            return pair

        @pl.when(c == 0)
        def _():
            lax.fori_loop(0, (nchunks + steal) // 2, make_pair(nchunks + steal),
                          0)

        @pl.when(c != 0)
        def _():
            lax.fori_loop(0, (nchunks - steal) // 2, make_pair(nchunks - steal),
                          0)

    return k(src_idx3, node_features)


NACC = 10240


def _sc_scatter_add(tgt_idx3, zz, zeros, nchunks):
    mesh = plsc.VectorSubcoreMesh(core_axis_name="c", subcore_axis_name="s")
    stripe = NACC // 16
    per_w = nchunks * CHUNK

    @functools.partial(
        pl.kernel, mesh=mesh,
        out_type=jax.ShapeDtypeStruct((2, NACC, OUT_C), jnp.float32),
        scratch_types=[
            pltpu.VMEM((nchunks, CHUNK), jnp.int32),
            pltpu.VMEM((CHUNK, OUT_C), jnp.float32),
            pltpu.VMEM((CHUNK, OUT_C), jnp.float32),
            pltpu.VMEM_SHARED((NACC, OUT_C), jnp.float32),
            pltpu.SemaphoreType.DMA,
            pltpu.SemaphoreType.DMA,
        ],
        compiler_params=pltpu.CompilerParams(use_tc_tiling_on_sc=True),
    )
    def k(idx_hbm, z_hbm, zeros_hbm, out_hbm, idx_v, z0, z1, acc, sem0, sem1):
        c = lax.axis_index("c")
        s = lax.axis_index("s")
        wid = s * 2 + c
        pltpu.sync_copy(zeros_hbm.at[pl.ds(s * stripe, stripe)],
                        acc.at[pl.ds(s * stripe, stripe)])
        pltpu.sync_copy(idx_hbm.at[wid], idx_v)
        plsc.subcore_barrier()
        base = wid * per_w
        bufs = (z0, z1)
        sems = (sem0, sem1)
        pltpu.async_copy(z_hbm.at[pl.ds(base, CHUNK)], z0, sem0)
        pltpu.async_copy(z_hbm.at[pl.ds(base + CHUNK, CHUNK)], z1, sem1)

        def pair(t, carry):
            for bslot in range(2):
                j = t * 2 + bslot
                buf, sem = bufs[bslot], sems[bslot]
                pltpu.make_async_copy(z_hbm.at[pl.ds(base, CHUNK)], buf,
                                      sem).wait()
                pltpu.sync_copy(buf, acc.at[idx_v.at[j]], add=True)

                @pl.when(j + 2 < nchunks)
                def _():
                    pltpu.async_copy(
                        z_hbm.at[pl.ds(base + (j + 2) * CHUNK, CHUNK)],
                        buf, sem)
            return carry

        lax.fori_loop(0, nchunks // 2, pair, 0)
        plsc.subcore_barrier()
        pltpu.sync_copy(acc.at[pl.ds(s * stripe, stripe)],
                        out_hbm.at[c, pl.ds(s * stripe, stripe)])

    return k(tgt_idx3, zz, zeros)


PHASE_CHUNKS = CHUNKS_PER_W // 2
PHASE_EDGES = NUM_WORKERS * PHASE_CHUNKS * CHUNK


def _combine_body(pa0_ref, pa1_ref, pb0_ref, pb1_ref, b_ref, o_ref):
    o_ref[...] = ((pa0_ref[0] + pa1_ref[0]) + (pb0_ref[0] + pb1_ref[0])
                  + b_ref[...])


def _combine(pa, pb, b):
    nb = 5
    blk = N_NODES_C // nb
    return pl.pallas_call(
        _combine_body,
        grid=(nb,),
        in_specs=[
            pl.BlockSpec((1, blk, OUT_C), lambda i: (0, i, 0)),
            pl.BlockSpec((1, blk, OUT_C), lambda i: (1, i, 0)),
            pl.BlockSpec((1, blk, OUT_C), lambda i: (0, i, 0)),
            pl.BlockSpec((1, blk, OUT_C), lambda i: (1, i, 0)),
            pl.BlockSpec((1, OUT_C), lambda i: (0, 0)),
        ],
        out_specs=pl.BlockSpec((blk, OUT_C), lambda i: (i, 0)),
        out_shape=jax.ShapeDtypeStruct((N_NODES_C, OUT_C), jnp.float32),
        interpret=_INTERPRET,
    )(pa, pa, pb, pb, b.reshape(1, OUT_C))


def _edge_matmul_body(g_ref, evt_ref, w2t_ref, tconst_ref, econst_ref, z_ref):
    x = evt_ref[0:1, :]
    y = evt_ref[1:2, :]
    z = evt_ref[2:3, :]
    n2 = x * x + y * y + z * z
    inv = jax.lax.rsqrt(jnp.maximum(n2, 1e-24))
    x = x * inv
    y = y * inv
    z = z * inv
    x2, y2, z2 = x * x, y * y, z * z
    sh = [
        0.28209479177387814 * jnp.ones_like(x),
        0.4886025119029199 * y,
        0.4886025119029199 * z,
        0.4886025119029199 * x,
        1.0925484305920792 * x * y,
        1.0925484305920792 * y * z,
        0.31539156525252005 * (3.0 * z2 - 1.0),
        1.0925484305920792 * x * z,
        0.5462742152960396 * (x2 - y2),
        0.5900435899266435 * y * (3.0 * x2 - y2),
        2.890611442640554 * x * y * z,
        0.4570457994644658 * y * (5.0 * z2 - 1.0),
        0.3731763325901154 * z * (5.0 * z2 - 3.0),
        0.4570457994644658 * x * (5.0 * z2 - 1.0),
        1.445305721320277 * z * (x2 - y2),
        0.5900435899266435 * x * (x2 - 3.0 * y2),
    ]
    sht = jnp.concatenate(sh, axis=0)
    shc = sht.T.astype(jnp.bfloat16)
    dims = (((1,), (0,)), ((), ()))
    half = EDGE_BLOCK // 4
    for h in range(4):
        rows = pl.ds(h * half, half)
        g = g_ref[rows, :].astype(jnp.bfloat16)
        gexp = jax.lax.dot_general(g, tconst_ref[...], dimension_numbers=dims,
                                   preferred_element_type=jnp.float32)
        shexp = jax.lax.dot_general(shc[h * half:(h + 1) * half, :],
                                    econst_ref[...], dimension_numbers=dims,
                                    preferred_element_type=jnp.float32)
        msgs = gexp.astype(jnp.bfloat16) * shexp.astype(jnp.bfloat16)
        z_ref[rows, :] = jax.lax.dot_general(
            msgs, w2t_ref[...],
            dimension_numbers=dims,
            preferred_element_type=jnp.float32,
        )


def _edge_matmul(gathered, evt, w2t, tconst, econst, phase):
    n_rows = gathered.shape[0]
    grid = n_rows // EDGE_BLOCK
    return pl.pallas_call(
        _edge_matmul_body,
        grid=(grid,),
        in_specs=[
            pl.BlockSpec((EDGE_BLOCK, HIDDEN_C), lambda i: (i, 0)),
            pl.BlockSpec((3, EDGE_BLOCK), lambda i: (0, 2 * i + phase)),
            pl.BlockSpec((HIDDEN_C * 16, OUT_C), lambda i: (0, 0)),
            pl.BlockSpec((HIDDEN_C, HIDDEN_C * 16), lambda i: (0, 0)),
            pl.BlockSpec((16, HIDDEN_C * 16), lambda i: (0, 0)),
        ],
        out_specs=pl.BlockSpec((EDGE_BLOCK, OUT_C), lambda i: (i, 0)),
        out_shape=jax.ShapeDtypeStruct((n_rows, OUT_C), jnp.float32),
        interpret=_INTERPRET,
    )(gathered, evt, w2t, tconst, econst)


def kernel(node_features, edge_vectors, edge_index, W, b):
    n_atoms = node_features.shape[0]
    n_edges = edge_index.shape[1]
    sources = edge_index[0]
    targets = edge_index[1]
    w2t = W.reshape(OUT_C, HIDDEN_C, 16).transpose(2, 1, 0).reshape(
        16 * HIDDEN_C, OUT_C).astype(jnp.bfloat16)
    src_pad = jnp.pad(sources, (0, E_PAD - n_edges)).reshape(
        NUM_WORKERS, CHUNKS_PER_W, CHUNK)
    tgt_pad = jnp.pad(targets, (0, E_PAD - n_edges),
                      constant_values=n_atoms).reshape(
        NUM_WORKERS, CHUNKS_PER_W, CHUNK)
    evt = jnp.pad(edge_vectors, ((0, E_PAD - n_edges), (0, 0))).T
    lm = jnp.arange(16 * HIDDEN_C) // HIDDEN_C
    ch = jnp.arange(16 * HIDDEN_C) % HIDDEN_C
    tconst = (jnp.arange(HIDDEN_C)[:, None] == ch[None, :]).astype(jnp.bfloat16)
    econst = (jnp.arange(16)[:, None] == lm[None, :]).astype(jnp.bfloat16)
    zeros = jnp.zeros((NACC, OUT_C), jnp.float32)
    ga = _sc_gather(src_pad[:, :PHASE_CHUNKS], node_features, PHASE_CHUNKS)
    za = _edge_matmul(ga, evt, w2t, tconst, econst, 0)
    gb = _sc_gather(src_pad[:, PHASE_CHUNKS:], node_features, PHASE_CHUNKS)
    pa = _sc_scatter_add(tgt_pad[:, :PHASE_CHUNKS], za, zeros, PHASE_CHUNKS)
    zb = _edge_matmul(gb, evt, w2t, tconst, econst, 1)
    pb = _sc_scatter_add(tgt_pad[:, PHASE_CHUNKS:], zb, zeros, PHASE_CHUNKS)
    return _combine(pa, pb, b)
